# baseline (device time: 446778 ns/iter reference)
import jax
import jax.numpy as jnp
from jax import lax
from jax.experimental import pallas as pl
from jax.experimental.pallas import tpu as pltpu

N = 16
ROWS = 1024
COLS = 512
D_ROWS = 8
D_COLS = 128


def _body(x_ref, d_ref, out_x, out_d, x_send, x_recv, d_send, d_recv):
    me = lax.axis_index("i")
    left = lax.rem(me + (N - 1), N)
    right = lax.rem(me + 1, N)

    barrier = pltpu.get_barrier_semaphore()
    pl.semaphore_signal(barrier, inc=1, device_id=(left,),
                        device_id_type=pl.DeviceIdType.MESH)
    pl.semaphore_signal(barrier, inc=1, device_id=(right,),
                        device_id_type=pl.DeviceIdType.MESH)
    pl.semaphore_wait(barrier, 2)

    out_x[pl.ds(me * ROWS, ROWS), :] = x_ref[...]
    out_d[pl.ds(me * D_ROWS, D_ROWS), :] = d_ref[...]

    sends = []
    recvs = []
    for h in range(N - 1):
        cs = lax.rem(me - h + N, N)
        cr = lax.rem(me - 1 - h + 2 * N, N)
        send_x = pltpu.make_async_remote_copy(
            src_ref=out_x.at[pl.ds(cs * ROWS, ROWS), :],
            dst_ref=out_x.at[pl.ds(cs * ROWS, ROWS), :],
            send_sem=x_send.at[h], recv_sem=x_recv.at[h],
            device_id=(right,), device_id_type=pl.DeviceIdType.MESH,
        )
        send_d = pltpu.make_async_remote_copy(
            src_ref=out_d.at[pl.ds(cs * D_ROWS, D_ROWS), :],
            dst_ref=out_d.at[pl.ds(cs * D_ROWS, D_ROWS), :],
            send_sem=d_send.at[h], recv_sem=d_recv.at[h],
            device_id=(right,), device_id_type=pl.DeviceIdType.MESH,
        )
        recv_x = pltpu.make_async_remote_copy(
            src_ref=out_x.at[pl.ds(cr * ROWS, ROWS), :],
            dst_ref=out_x.at[pl.ds(cr * ROWS, ROWS), :],
            send_sem=x_send.at[h], recv_sem=x_recv.at[h],
            device_id=(left,), device_id_type=pl.DeviceIdType.MESH,
        )
        recv_d = pltpu.make_async_remote_copy(
            src_ref=out_d.at[pl.ds(cr * D_ROWS, D_ROWS), :],
            dst_ref=out_d.at[pl.ds(cr * D_ROWS, D_ROWS), :],
            send_sem=d_send.at[h], recv_sem=d_recv.at[h],
            device_id=(left,), device_id_type=pl.DeviceIdType.MESH,
        )
        if h > 0:
            recvs[h - 1][0].wait_recv()
            recvs[h - 1][1].wait_recv()
        send_x.start()
        send_d.start()
        sends.append((send_x, send_d))
        recvs.append((recv_x, recv_d))

    recvs[-1][0].wait_recv()
    recvs[-1][1].wait_recv()
    for sx, sd in sends:
        sx.wait_send()
        sd.wait_send()


def kernel(x, dest):
    x_bf = x.astype(jnp.bfloat16)
    d2 = dest.reshape(D_ROWS, D_COLS)

    gx, gd = pl.pallas_call(
        _body,
        out_shape=[
            jax.ShapeDtypeStruct((N * ROWS, COLS), jnp.bfloat16),
            jax.ShapeDtypeStruct((N * D_ROWS, D_COLS), jnp.int32),
        ],
        in_specs=[
            pl.BlockSpec(memory_space=pltpu.VMEM),
            pl.BlockSpec(memory_space=pltpu.VMEM),
        ],
        out_specs=[
            pl.BlockSpec(memory_space=pltpu.VMEM),
            pl.BlockSpec(memory_space=pltpu.VMEM),
        ],
        scratch_shapes=[
            pltpu.SemaphoreType.DMA((N - 1,)),
            pltpu.SemaphoreType.DMA((N - 1,)),
            pltpu.SemaphoreType.DMA((N - 1,)),
            pltpu.SemaphoreType.DMA((N - 1,)),
        ],
        compiler_params=pltpu.CompilerParams(collective_id=0),
    )(x_bf, d2)

    me = lax.axis_index("i")
    mask = gd.reshape(-1) == me
    csum = jnp.cumsum(mask.astype(jnp.int32))
    idx = jnp.searchsorted(csum, jnp.arange(1, ROWS + 1, dtype=jnp.int32),
                           side="left")
    return gx[idx].astype(jnp.float32)


# device time: 54569 ns/iter; 8.1874x vs baseline; 8.1874x over previous
import jax
import jax.numpy as jnp
from jax import lax
from jax.experimental import pallas as pl
from jax.experimental.pallas import tpu as pltpu

N = 16
ROWS = 1024
COLS = 512
D_ROWS = 8
D_COLS = 128


def _body(x_ref, counts_ref, dest_vm, rank_vm, dest_sm,
          out_ref, c_buf, p_sm, p_vm,
          bf_send, bf_recv, row_send, row_recv, copy_sem):
    me = lax.axis_index("i")

    barrier = pltpu.get_barrier_semaphore()
    for k in range(N):
        @pl.when(k != me)
        def _():
            pl.semaphore_signal(barrier, inc=1, device_id=(k,),
                                device_id_type=pl.DeviceIdType.MESH)
    pl.semaphore_wait(barrier, N - 1)

    c_buf[pl.ds(me, 1), :] = counts_ref[pl.ds(0, 1), :]
    for j in range(4):
        sz = 1 << j
        a = jnp.bitwise_and(me, ~(sz - 1) & (N - 1))
        b = jnp.bitwise_xor(a, sz)
        partner = jnp.bitwise_xor(me, sz)
        send = pltpu.make_async_remote_copy(
            src_ref=c_buf.at[pl.ds(a, sz), :],
            dst_ref=c_buf.at[pl.ds(a, sz), :],
            send_sem=bf_send.at[j], recv_sem=bf_recv.at[j],
            device_id=(partner,), device_id_type=pl.DeviceIdType.MESH,
        )
        recv = pltpu.make_async_remote_copy(
            src_ref=c_buf.at[pl.ds(b, sz), :],
            dst_ref=c_buf.at[pl.ds(b, sz), :],
            send_sem=bf_send.at[j], recv_sem=bf_recv.at[j],
            device_id=(partner,), device_id_type=pl.DeviceIdType.MESH,
        )
        send.start()
        recv.wait_recv()
        send.wait_send()

    c = c_buf[...]
    src_id = lax.broadcasted_iota(jnp.int32, (N, D_COLS), 0)
    base_vec = jnp.sum(jnp.where(src_id < me, c, 0), axis=0)
    d2 = dest_vm[...]
    p2 = rank_vm[...]
    for d in range(N):
        p2 = p2 + jnp.where(d2 == d, base_vec[d], 0)
    p_vm[...] = p2

    cp = pltpu.make_async_copy(p_vm, p_sm, copy_sem)
    cp.start()
    cp.wait()

    def issue(k, _):
        j = k // D_COLS
        l = lax.rem(k, D_COLS)
        d = dest_sm[j, l]
        p = p_sm[j, l]
        rdma = pltpu.make_async_remote_copy(
            src_ref=x_ref.at[pl.ds(pl.multiple_of(k * COLS, COLS), COLS)],
            dst_ref=out_ref.at[pl.ds(pl.multiple_of(p * COLS, COLS), COLS)],
            send_sem=row_send, recv_sem=row_recv,
            device_id=(d,), device_id_type=pl.DeviceIdType.MESH,
        )
        rdma.start()
        return 0

    lax.fori_loop(0, ROWS, issue, 0)

    def drain_send(k, _):
        pltpu.make_async_remote_copy(
            src_ref=x_ref.at[pl.ds(0, COLS)],
            dst_ref=out_ref.at[pl.ds(0, COLS)],
            send_sem=row_send, recv_sem=row_recv,
            device_id=(me,), device_id_type=pl.DeviceIdType.MESH,
        ).wait_send()
        return 0

    def drain_recv(k, _):
        pltpu.make_async_remote_copy(
            src_ref=x_ref.at[pl.ds(0, COLS)],
            dst_ref=out_ref.at[pl.ds(0, COLS)],
            send_sem=row_send, recv_sem=row_recv,
            device_id=(me,), device_id_type=pl.DeviceIdType.MESH,
        ).wait_recv()
        return 0

    lax.fori_loop(0, ROWS, drain_send, 0)
    lax.fori_loop(0, ROWS, drain_recv, 0)


def kernel(x, dest):
    x_bf = x.astype(jnp.bfloat16)
    oh = (dest[:, None] == jnp.arange(N, dtype=dest.dtype)).astype(jnp.int32)
    rank = jnp.sum((jnp.cumsum(oh, axis=0) - oh) * oh, axis=1)
    counts = jnp.sum(oh, axis=0)
    counts8 = jnp.zeros((D_ROWS, D_COLS), jnp.int32).at[0, :N].set(counts)
    d2 = dest.reshape(D_ROWS, D_COLS)
    r2 = rank.reshape(D_ROWS, D_COLS).astype(jnp.int32)

    out = pl.pallas_call(
        _body,
        out_shape=jax.ShapeDtypeStruct((ROWS * COLS,), jnp.bfloat16),
        in_specs=[
            pl.BlockSpec(memory_space=pltpu.MemorySpace.HBM),
            pl.BlockSpec(memory_space=pltpu.VMEM),
            pl.BlockSpec(memory_space=pltpu.VMEM),
            pl.BlockSpec(memory_space=pltpu.VMEM),
            pl.BlockSpec(memory_space=pltpu.SMEM),
        ],
        out_specs=pl.BlockSpec(memory_space=pltpu.MemorySpace.HBM),
        scratch_shapes=[
            pltpu.VMEM((N, D_COLS), jnp.int32),
            pltpu.SMEM((D_ROWS, D_COLS), jnp.int32),
            pltpu.VMEM((D_ROWS, D_COLS), jnp.int32),
            pltpu.SemaphoreType.DMA((4,)),
            pltpu.SemaphoreType.DMA((4,)),
            pltpu.SemaphoreType.DMA,
            pltpu.SemaphoreType.DMA,
            pltpu.SemaphoreType.DMA,
        ],
        compiler_params=pltpu.CompilerParams(collective_id=0),
    )(x_bf.reshape(-1), counts8, d2, r2, d2)

    return out.reshape(ROWS, COLS).astype(jnp.float32)


# device time: 39210 ns/iter; 11.3945x vs baseline; 1.3917x over previous
import jax
import jax.numpy as jnp
from jax import lax
from jax.experimental import pallas as pl
from jax.experimental.pallas import tpu as pltpu

N = 16
ROWS = 1024
COLS = 512
D_ROWS = 8
D_COLS = 128


def _body(x_ref, counts_ref, dest_vm, rank_vm,
          out_ref, c_buf, w_sm, w_vm,
          cnt_send, cnt_recv, row_send, row_recv, copy_sem):
    me = lax.axis_index("i")

    barrier = pltpu.get_barrier_semaphore()
    for k in range(N):
        @pl.when(k != me)
        def _():
            pl.semaphore_signal(barrier, inc=1, device_id=(k,),
                                device_id_type=pl.DeviceIdType.MESH)
    pl.semaphore_wait(barrier, N - 1)

    c_buf[pl.ds(me, 1), :] = counts_ref[pl.ds(0, 1), :]
    for k in range(N):
        @pl.when(k != me)
        def _():
            pltpu.make_async_remote_copy(
                src_ref=c_buf.at[pl.ds(me, 1), :],
                dst_ref=c_buf.at[pl.ds(me, 1), :],
                send_sem=cnt_send, recv_sem=cnt_recv,
                device_id=(k,), device_id_type=pl.DeviceIdType.MESH,
            ).start()

    def cnt_dummy():
        return pltpu.make_async_remote_copy(
            src_ref=c_buf.at[pl.ds(0, 1), :],
            dst_ref=c_buf.at[pl.ds(0, 1), :],
            send_sem=cnt_send, recv_sem=cnt_recv,
            device_id=(me,), device_id_type=pl.DeviceIdType.MESH,
        )

    for _ in range(N - 1):
        cnt_dummy().wait_recv()

    c = c_buf[...]
    src_id = lax.broadcasted_iota(jnp.int32, (N, D_COLS), 0)
    base_vec = jnp.sum(jnp.where(src_id < me, c, 0), axis=0)
    d2 = dest_vm[...]
    p2 = rank_vm[...]
    for d in range(N):
        p2 = p2 + jnp.where(d2 == d, base_vec[d], 0)
    w_vm[...] = jnp.left_shift(p2, 4) + d2

    cp = pltpu.make_async_copy(w_vm, w_sm, copy_sem)
    cp.start()
    cp.wait()

    def outer(j, _):
        def inner(l, _):
            w = w_sm[j, l]
            p = lax.shift_right_logical(w, 4)
            d = lax.bitwise_and(w, 15)
            k = j * D_COLS + l
            pltpu.make_async_remote_copy(
                src_ref=x_ref.at[pl.ds(pl.multiple_of(k * COLS, COLS), COLS)],
                dst_ref=out_ref.at[pl.ds(pl.multiple_of(p * COLS, COLS), COLS)],
                send_sem=row_send, recv_sem=row_recv,
                device_id=(d,), device_id_type=pl.DeviceIdType.MESH,
            ).start()
            return 0
        lax.fori_loop(0, D_COLS, inner, 0)
        return 0

    lax.fori_loop(0, D_ROWS, outer, 0)

    full = pltpu.make_async_remote_copy(
        src_ref=x_ref.at[pl.ds(0, ROWS * COLS)],
        dst_ref=out_ref.at[pl.ds(0, ROWS * COLS)],
        send_sem=row_send, recv_sem=row_recv,
        device_id=(me,), device_id_type=pl.DeviceIdType.MESH,
    )
    full.wait_send()
    full.wait_recv()
    for _ in range(N - 1):
        cnt_dummy().wait_send()


def kernel(x, dest):
    x_bf = x.astype(jnp.bfloat16)
    oh = (dest[:, None] == jnp.arange(N, dtype=dest.dtype)).astype(jnp.int32)
    rank = jnp.sum((jnp.cumsum(oh, axis=0) - oh) * oh, axis=1)
    counts = jnp.sum(oh, axis=0)
    counts8 = jnp.zeros((D_ROWS, D_COLS), jnp.int32).at[0, :N].set(counts)
    d2 = dest.reshape(D_ROWS, D_COLS)
    r2 = rank.reshape(D_ROWS, D_COLS).astype(jnp.int32)

    out = pl.pallas_call(
        _body,
        out_shape=jax.ShapeDtypeStruct((ROWS * COLS,), jnp.bfloat16),
        in_specs=[
            pl.BlockSpec(memory_space=pltpu.MemorySpace.HBM),
            pl.BlockSpec(memory_space=pltpu.VMEM),
            pl.BlockSpec(memory_space=pltpu.VMEM),
            pl.BlockSpec(memory_space=pltpu.VMEM),
        ],
        out_specs=pl.BlockSpec(memory_space=pltpu.MemorySpace.HBM),
        scratch_shapes=[
            pltpu.VMEM((N, D_COLS), jnp.int32),
            pltpu.SMEM((D_ROWS, D_COLS), jnp.int32),
            pltpu.VMEM((D_ROWS, D_COLS), jnp.int32),
            pltpu.SemaphoreType.DMA,
            pltpu.SemaphoreType.DMA,
            pltpu.SemaphoreType.DMA,
            pltpu.SemaphoreType.DMA,
            pltpu.SemaphoreType.DMA,
        ],
        compiler_params=pltpu.CompilerParams(collective_id=0),
    )(x_bf.reshape(-1), counts8, d2, r2)

    return out.reshape(ROWS, COLS).astype(jnp.float32)
